# Initial kernel scaffold; baseline (speedup 1.0000x reference)
import jax
import jax.numpy as jnp
from jax.experimental import pallas as pl


def kernel(coords_xyz, grid):
    raise NotImplementedError("write your pallas kernel here")


# speedup vs baseline: 1.1057x; 1.1057x over previous
import functools

import jax
import jax.numpy as jnp
from jax import lax
from jax.experimental import pallas as pl
from jax.experimental.pallas import tpu as pltpu
from jax.experimental.pallas import tpu_sc as plsc

DIM_GRID = 128
DIM_PAYLOAD = 16
BATCH = 262144

NUM_WORKERS = 32
PTS_PER_WORKER = BATCH // NUM_WORKERS
BLK = 256
NUM_BLKS = PTS_PER_WORKER // BLK
GROUPS = BLK // 16
ROWS_PER_BLK = BLK * 8
CHUNK = 128
NUM_CHUNKS = ROWS_PER_BLK // CHUNK


def _make_sc_interp():
  mesh = plsc.VectorSubcoreMesh(core_axis_name="c", subcore_axis_name="s")

  @functools.partial(
      pl.kernel,
      out_type=jax.ShapeDtypeStruct((BATCH, DIM_PAYLOAD), jnp.float32),
      mesh=mesh,
      compiler_params=pltpu.CompilerParams(needs_layout_passes=False, use_tc_tiling_on_sc=False),
      scratch_types=[
          pltpu.VMEM((PTS_PER_WORKER * 3,), jnp.float32),
          pltpu.VMEM((GROUPS, CHUNK), jnp.int32),
          pltpu.VMEM((4, BLK), jnp.float32),
          pltpu.VMEM((ROWS_PER_BLK, DIM_PAYLOAD), jnp.float32),
          pltpu.VMEM((BLK, DIM_PAYLOAD), jnp.float32),
          pltpu.SemaphoreType.DMA,
      ],
  )
  def interp(coords_hbm, table_hbm, out_hbm, coords_v, idx2d, fm, rows,
             out_v, sem):
    wid = lax.axis_index("s") * 2 + lax.axis_index("c")
    base_pt = wid * PTS_PER_WORKER

    pltpu.async_copy(
        coords_hbm.at[pl.ds(base_pt * 3, PTS_PER_WORKER * 3)], coords_v, sem
    ).wait()

    iota16 = lax.iota(jnp.int32, 16)

    def do_block(b, carry):
      def compute_group(g, c2):
        p3 = (b * BLK + g * 16 + iota16) * 3
        x = plsc.load_gather(coords_v, [p3])
        y = plsc.load_gather(coords_v, [p3 + 1])
        z = plsc.load_gather(coords_v, [p3 + 2])

        cx = (x + 1.0) * (0.5 * (DIM_GRID - 1))
        cy = (y + 1.0) * (0.5 * (DIM_GRID - 1))
        cz = (z + 1.0) * (0.5 * (DIM_GRID - 1))
        ix = jnp.clip(cx.astype(jnp.int32), 0, DIM_GRID - 1)
        iy = jnp.clip(cy.astype(jnp.int32), 0, DIM_GRID - 1)
        iz = jnp.clip(cz.astype(jnp.int32), 0, DIM_GRID - 1)
        fx = cx - ix.astype(jnp.float32)
        fy = cy - iy.astype(jnp.float32)
        fz = cz - iz.astype(jnp.float32)
        hx = jnp.minimum(ix + 1, DIM_GRID - 1)
        hy = jnp.minimum(iy + 1, DIM_GRID - 1)
        hz = jnp.minimum(iz + 1, DIM_GRID - 1)

        one = jnp.full((16,), 1.0, jnp.float32)
        zero = jnp.zeros((16,), jnp.float32)
        valid = (
            (x >= -1.0) & (x <= 1.0)
            & (y >= -1.0) & (y <= 1.0)
            & (z >= -1.0) & (z <= 1.0)
        )
        m = jnp.where(valid, one, zero)

        xlo = ix * (DIM_GRID * DIM_GRID)
        xhi = hx * (DIM_GRID * DIM_GRID)
        ylo = iy * DIM_GRID
        yhi = hy * DIM_GRID
        a_ll = xlo + ylo
        a_lh = xlo + yhi
        a_hl = xhi + ylo
        a_hh = xhi + yhi
        idx2d[g, pl.ds(0 * 16, 16)] = a_ll + iz
        idx2d[g, pl.ds(1 * 16, 16)] = a_ll + hz
        idx2d[g, pl.ds(2 * 16, 16)] = a_lh + iz
        idx2d[g, pl.ds(3 * 16, 16)] = a_lh + hz
        idx2d[g, pl.ds(4 * 16, 16)] = a_hl + iz
        idx2d[g, pl.ds(5 * 16, 16)] = a_hl + hz
        idx2d[g, pl.ds(6 * 16, 16)] = a_hh + iz
        idx2d[g, pl.ds(7 * 16, 16)] = a_hh + hz

        fm[0, pl.ds(g * 16, 16)] = fx
        fm[1, pl.ds(g * 16, 16)] = fy
        fm[2, pl.ds(g * 16, 16)] = fz
        fm[3, pl.ds(g * 16, 16)] = m
        return c2

      lax.fori_loop(0, GROUPS, compute_group, 0)

      cps = []
      for j in range(NUM_CHUNKS):
        cps.append(
            pltpu.async_copy(
                table_hbm.at[idx2d.at[j]],
                rows.at[pl.ds(j * CHUNK, CHUNK), :],
                sem,
            )
        )
      for cp in cps:
        cp.wait()

      def acc_group(g, c2):
        fxv = fm[0, pl.ds(g * 16, 16)]
        fyv = fm[1, pl.ds(g * 16, 16)]
        fzv = fm[2, pl.ds(g * 16, 16)]
        mv = fm[3, pl.ds(g * 16, 16)]
        base_g = g * 128
        for pig in range(16):
          base_r = base_g + pig
          r0 = rows[base_r + 0 * 16]
          r1 = rows[base_r + 1 * 16]
          r2 = rows[base_r + 2 * 16]
          r3 = rows[base_r + 3 * 16]
          r4 = rows[base_r + 4 * 16]
          r5 = rows[base_r + 5 * 16]
          r6 = rows[base_r + 6 * 16]
          r7 = rows[base_r + 7 * 16]
          fx = fxv[pig]
          fy = fyv[pig]
          fz = fzv[pig]
          m = mv[pig]
          a00 = r0 + fz * (r1 - r0)
          a01 = r2 + fz * (r3 - r2)
          a10 = r4 + fz * (r5 - r4)
          a11 = r6 + fz * (r7 - r6)
          b0 = a00 + fy * (a01 - a00)
          b1 = a10 + fy * (a11 - a10)
          o = b0 + fx * (b1 - b0)
          out_v[g * 16 + pig] = o * m
        return c2

      lax.fori_loop(0, GROUPS, acc_group, 0)

      pltpu.sync_copy(
          out_v, out_hbm.at[pl.ds(base_pt + b * BLK, BLK), :]
      )
      return carry

    lax.fori_loop(0, NUM_BLKS, do_block, 0)

  return interp


_sc_interp = _make_sc_interp()


@jax.jit
def kernel(coords_xyz, grid):
  table = grid.reshape(DIM_GRID * DIM_GRID * DIM_GRID, DIM_PAYLOAD)
  return _sc_interp(coords_xyz.reshape(-1), table)
